# Initial kernel scaffold; baseline (speedup 1.0000x reference)
import jax
import jax.numpy as jnp
from jax.experimental import pallas as pl


def kernel(text, W, b):
    raise NotImplementedError("write your pallas kernel here")


# speedup vs baseline: 23.3099x; 23.3099x over previous
import functools

import jax
import jax.numpy as jnp
from jax import lax
from jax.experimental import pallas as pl
from jax.experimental.pallas import tpu as pltpu
from jax.experimental.pallas import tpu_sc as plsc

SEQ = 200
BATCH = 1024
VOCAB = 100000


def _make_kernel():
    nc, ns, nl = 2, 16, 16
    nw = nc * ns
    b_per_w = BATCH // nw
    groups = b_per_w // nl

    mesh = plsc.VectorSubcoreMesh(core_axis_name="c", subcore_axis_name="s")

    @functools.partial(
        pl.kernel,
        mesh=mesh,
        out_type=jax.ShapeDtypeStruct((BATCH,), jnp.float32),
        compiler_params=pltpu.CompilerParams(needs_layout_passes=False),
        scratch_types=[
            pltpu.VMEM((VOCAB,), jnp.float32),
            pltpu.VMEM((SEQ * b_per_w,), jnp.int32),
            pltpu.VMEM((b_per_w,), jnp.float32),
            pltpu.VMEM((nl,), jnp.float32),
        ],
    )
    def k(text_hbm, w_hbm, b_hbm, out_hbm, w_v, tok_v, out_v, bias_v):
        wid = lax.axis_index("s") * nc + lax.axis_index("c")
        base = wid * b_per_w
        pltpu.sync_copy(w_hbm, w_v)
        pltpu.sync_copy(
            text_hbm.at[pl.ds(wid * SEQ * b_per_w, SEQ * b_per_w)], tok_v
        )
        pltpu.sync_copy(b_hbm, bias_v)
        bias = bias_v[...]
        for g in range(groups):
            def body(t, acc):
                idx = tok_v[pl.ds(t * b_per_w + g * nl, nl)]
                return acc + plsc.load_gather(w_v, [idx])
            acc = lax.fori_loop(0, SEQ, body, jnp.zeros((nl,), jnp.float32))
            out_v[pl.ds(g * nl, nl)] = acc + bias
        pltpu.sync_copy(out_v, out_hbm.at[pl.ds(base, b_per_w)])

    return k


def kernel(text, W, b):
    text_i32 = (
        text.astype(jnp.int32)
        .reshape(SEQ, BATCH // 32, 32)
        .transpose(1, 0, 2)
        .reshape(SEQ * BATCH)
    )
    w_flat = W.reshape(VOCAB).astype(jnp.float32)
    b_vec = jnp.broadcast_to(b.astype(jnp.float32), (16,))
    out = _make_kernel()(text_i32, w_flat, b_vec)
    return out.reshape(BATCH, 1)


# speedup vs baseline: 25.6113x; 1.0987x over previous
import functools

import jax
import jax.numpy as jnp
from jax import lax
from jax.experimental import pallas as pl
from jax.experimental.pallas import tpu as pltpu
from jax.experimental.pallas import tpu_sc as plsc

SEQ = 200
BATCH = 1024
VOCAB = 100000


def _make_kernel():
    nc, ns, nl = 2, 16, 16
    nw = nc * ns
    b_per_w = BATCH // nw
    groups = b_per_w // nl

    mesh = plsc.VectorSubcoreMesh(core_axis_name="c", subcore_axis_name="s")

    @functools.partial(
        pl.kernel,
        mesh=mesh,
        out_type=jax.ShapeDtypeStruct((BATCH,), jnp.float32),
        compiler_params=pltpu.CompilerParams(needs_layout_passes=False),
        scratch_types=[
            pltpu.VMEM((VOCAB,), jnp.float32),
            pltpu.VMEM((SEQ * b_per_w,), jnp.int32),
            pltpu.VMEM((b_per_w,), jnp.float32),
            pltpu.VMEM((nl,), jnp.float32),
            pltpu.SemaphoreType.DMA,
            pltpu.SemaphoreType.DMA,
        ],
    )
    def k(text_hbm, w_hbm, b_hbm, out_hbm, w_v, tok_v, out_v, bias_v,
          sem_w, sem_t):
        wid = lax.axis_index("s") * nc + lax.axis_index("c")
        base = wid * b_per_w
        cp_w = pltpu.async_copy(w_hbm, w_v, sem_w)
        cp_t = pltpu.async_copy(
            text_hbm.at[pl.ds(wid * SEQ * b_per_w, SEQ * b_per_w)], tok_v,
            sem_t,
        )
        pltpu.sync_copy(b_hbm, bias_v)
        bias = bias_v[...]
        cp_t.wait()
        cp_w.wait()

        def body(t, accs):
            a0, a1 = accs
            i0 = tok_v[pl.ds(t * b_per_w, nl)]
            i1 = tok_v[pl.ds(t * b_per_w + nl, nl)]
            return (a0 + plsc.load_gather(w_v, [i0]),
                    a1 + plsc.load_gather(w_v, [i1]))

        zero = jnp.zeros((nl,), jnp.float32)
        a0, a1 = lax.fori_loop(0, SEQ, body, (zero, zero), unroll=8)
        out_v[pl.ds(0, nl)] = a0 + bias
        out_v[pl.ds(nl, nl)] = a1 + bias
        pltpu.sync_copy(out_v, out_hbm.at[pl.ds(base, b_per_w)])

    return k


def kernel(text, W, b):
    text_i32 = (
        text.astype(jnp.int32)
        .reshape(SEQ, BATCH // 32, 32)
        .transpose(1, 0, 2)
        .reshape(SEQ * BATCH)
    )
    w_flat = W.reshape(VOCAB).astype(jnp.float32)
    b_vec = jnp.broadcast_to(b.astype(jnp.float32), (16,))
    out = _make_kernel()(text_i32, w_flat, b_vec)
    return out.reshape(BATCH, 1)


# speedup vs baseline: 28.0538x; 1.0954x over previous
import functools

import jax
import jax.numpy as jnp
from jax import lax
from jax.experimental import pallas as pl
from jax.experimental.pallas import tpu as pltpu
from jax.experimental.pallas import tpu_sc as plsc

SEQ = 200
BATCH = 1024
VOCAB = 100000


def _make_kernel():
    nc, ns, nl = 2, 16, 16
    nw = nc * ns
    b_per_w = BATCH // nw

    mesh = plsc.VectorSubcoreMesh(core_axis_name="c", subcore_axis_name="s")

    @functools.partial(
        pl.kernel,
        mesh=mesh,
        out_type=jax.ShapeDtypeStruct((BATCH,), jnp.float32),
        compiler_params=pltpu.CompilerParams(
            needs_layout_passes=False, use_tc_tiling_on_sc=False
        ),
        scratch_types=[
            pltpu.VMEM((VOCAB,), jnp.float32),
            pltpu.VMEM((SEQ, b_per_w), jnp.int32),
            pltpu.VMEM((b_per_w,), jnp.float32),
            pltpu.VMEM((nl,), jnp.float32),
            pltpu.SemaphoreType.DMA,
            pltpu.SemaphoreType.DMA,
        ],
    )
    def k(text_hbm, w_hbm, b_hbm, out_hbm, w_v, tok_v, out_v, bias_v,
          sem_w, sem_t):
        wid = lax.axis_index("s") * nc + lax.axis_index("c")
        base = wid * b_per_w
        cp_w = pltpu.async_copy(w_hbm.at[0], w_v, sem_w)
        cp_t = pltpu.async_copy(text_hbm.at[:, pl.ds(base, b_per_w)], tok_v,
                                sem_t)
        pltpu.sync_copy(b_hbm, bias_v.at[pl.ds(0, 1)])
        bias = bias_v[...][0]
        cp_t.wait()
        cp_w.wait()

        def body(t, accs):
            a0, a1 = accs
            i0 = tok_v[t, pl.ds(0, nl)]
            i1 = tok_v[t, pl.ds(nl, nl)]
            return (a0 + plsc.load_gather(w_v, [i0]),
                    a1 + plsc.load_gather(w_v, [i1]))

        zero = jnp.zeros((nl,), jnp.float32)
        a0, a1 = lax.fori_loop(0, SEQ, body, (zero, zero), unroll=8)
        out_v[pl.ds(0, nl)] = a0 + bias
        out_v[pl.ds(nl, nl)] = a1 + bias
        pltpu.sync_copy(out_v, out_hbm.at[pl.ds(base, b_per_w)])

    return k


def kernel(text, W, b):
    out = _make_kernel()(text.astype(jnp.int32), W, b)
    return out.reshape(BATCH, 1)
